# baseline (speedup 1.0000x reference)
import functools

import jax
import jax.numpy as jnp
from jax import lax
from jax.experimental import pallas as pl
from jax.experimental.pallas import tpu as pltpu
from jax.experimental.pallas import tpu_sc as plsc

D_MODEL = 64
SCALE = 8.0

B_TOTAL = 4096 * 200
NUM_CORES = 2
NUM_SUBCORES = 16
NW = NUM_CORES * NUM_SUBCORES
BPW = B_TOTAL // NW
CHUNK = 128
NCHUNK = BPW // CHUNK


def _build():
  mesh = plsc.VectorSubcoreMesh(core_axis_name="c", subcore_axis_name="s")

  @functools.partial(
      pl.kernel,
      mesh=mesh,
      out_type=jax.ShapeDtypeStruct((B_TOTAL, D_MODEL), jnp.float32),
      scratch_types=[
          pltpu.VMEM((CHUNK,), jnp.int32),
          pltpu.VMEM((CHUNK, D_MODEL), jnp.float32),
          pltpu.SemaphoreType.DMA,
      ],
      compiler_params=pltpu.CompilerParams(use_tc_tiling_on_sc=False),
  )
  def emb(idx_hbm, table_hbm, out_hbm, idx_v, rows_v, sem):
    wid = lax.axis_index("s") * NUM_CORES + lax.axis_index("c")
    base = wid * BPW

    def chunk_body(g, carry):
      off = base + g * CHUNK
      pltpu.sync_copy(idx_hbm.at[pl.ds(off, CHUNK)], idx_v)
      pltpu.async_copy(table_hbm.at[idx_v], rows_v, sem).wait()

      def scale_body(r, c2):
        for j in range(D_MODEL // 16):
          s = pl.ds(j * 16, 16)
          rows_v[r, s] = rows_v[r, s] * SCALE
        return c2

      lax.fori_loop(0, CHUNK, scale_body, 0)
      pltpu.sync_copy(rows_v, out_hbm.at[pl.ds(off, CHUNK)])
      return carry

    lax.fori_loop(0, NCHUNK, chunk_body, 0)

  return emb


_emb = _build()


@jax.jit
def kernel(x, lut):
  flat = x.reshape(-1)
  out = _emb(flat, lut)
  return out.reshape(x.shape + (D_MODEL,))


# speedup vs baseline: 1.1331x; 1.1331x over previous
import functools

import jax
import jax.numpy as jnp
from jax import lax
from jax.experimental import pallas as pl
from jax.experimental.pallas import tpu as pltpu
from jax.experimental.pallas import tpu_sc as plsc

D_MODEL = 64
SCALE = 8.0

B_TOTAL = 4096 * 200
NUM_CORES = 2
NUM_SUBCORES = 16
NW = NUM_CORES * NUM_SUBCORES
BPW = B_TOTAL // NW
CHUNK = 128
NCHUNK = BPW // CHUNK
NBUF = 4


def _build():
  mesh = plsc.VectorSubcoreMesh(core_axis_name="c", subcore_axis_name="s")

  @functools.partial(
      pl.kernel,
      mesh=mesh,
      out_type=jax.ShapeDtypeStruct((B_TOTAL, D_MODEL), jnp.float32),
      scratch_types=[
          pltpu.VMEM((NBUF, CHUNK), jnp.int32),
          pltpu.VMEM((NBUF, CHUNK, D_MODEL), jnp.float32),
          [pltpu.SemaphoreType.DMA] * NBUF,
          [pltpu.SemaphoreType.DMA] * NBUF,
      ],
      compiler_params=pltpu.CompilerParams(use_tc_tiling_on_sc=False),
  )
  def emb(idx_hbm, table_hbm, out_hbm, idx_v, rows_v, gsems, ssems):
    wid = lax.axis_index("s") * NUM_CORES + lax.axis_index("c")
    base = wid * BPW

    def start_gather(b, off):
      pltpu.sync_copy(idx_hbm.at[pl.ds(off, CHUNK)], idx_v.at[b])
      pltpu.make_async_copy(
          table_hbm.at[idx_v.at[b]], rows_v.at[b], gsems[b]
      ).start()

    for b in range(NBUF):
      start_gather(b, base + b * CHUNK)

    def scale_buf(b):
      def body(r, c2):
        for j in range(D_MODEL // 16):
          s = pl.ds(j * 16, 16)
          rows_v[b, r, s] = rows_v[b, r, s] * SCALE
        return c2

      lax.fori_loop(0, CHUNK, body, 0)

    def outer(i, carry):
      for b in range(NBUF):
        g = i * NBUF + b
        off = base + g * CHUNK
        pltpu.make_async_copy(
            table_hbm.at[idx_v.at[b]], rows_v.at[b], gsems[b]
        ).wait()
        scale_buf(b)
        pltpu.make_async_copy(
            rows_v.at[b], out_hbm.at[pl.ds(off, CHUNK)], ssems[b]
        ).start()

        @pl.when(g + NBUF < NCHUNK)
        def _():
          pltpu.make_async_copy(
              rows_v.at[b], out_hbm.at[pl.ds(off, CHUNK)], ssems[b]
          ).wait()
          start_gather(b, off + NBUF * CHUNK)

      return carry

    lax.fori_loop(0, NCHUNK // NBUF, outer, 0)

    for b in range(NBUF):
      pltpu.make_async_copy(
          rows_v.at[b], out_hbm.at[pl.ds(base, CHUNK)], ssems[b]
      ).wait()

  return emb


_emb = _build()


@jax.jit
def kernel(x, lut):
  out = _emb(x.reshape(-1), lut)
  return out.reshape(x.shape + (D_MODEL,))


# speedup vs baseline: 1.2724x; 1.1230x over previous
import functools

import jax
import jax.numpy as jnp
from jax import lax
from jax.experimental import pallas as pl
from jax.experimental.pallas import tpu as pltpu
from jax.experimental.pallas import tpu_sc as plsc

D_MODEL = 64
SCALE = 8.0

B_TOTAL = 4096 * 200
NUM_CORES = 2
NUM_SUBCORES = 16
NW = NUM_CORES * NUM_SUBCORES
BPW = B_TOTAL // NW
CHUNK = 128
NCHUNK = BPW // CHUNK
NBUF = 6
ROW_UNROLL = 16


def _build():
  mesh = plsc.VectorSubcoreMesh(core_axis_name="c", subcore_axis_name="s")

  @functools.partial(
      pl.kernel,
      mesh=mesh,
      out_type=jax.ShapeDtypeStruct((B_TOTAL, D_MODEL), jnp.float32),
      scratch_types=[
          pltpu.VMEM((NCHUNK, CHUNK), jnp.int32),
          pltpu.VMEM((NBUF, CHUNK, D_MODEL), jnp.float32),
          [pltpu.SemaphoreType.DMA] * NBUF,
          [pltpu.SemaphoreType.DMA] * NBUF,
      ],
      compiler_params=pltpu.CompilerParams(use_tc_tiling_on_sc=False),
  )
  def emb(idx_hbm, table_hbm, out_hbm, idx_v, rows_v, gsems, ssems):
    wid = lax.axis_index("s") * NUM_CORES + lax.axis_index("c")
    base = wid * BPW

    pltpu.sync_copy(idx_hbm.at[pl.ds(wid * NCHUNK, NCHUNK)], idx_v)

    def start_gather(b, g):
      pltpu.make_async_copy(
          table_hbm.at[idx_v.at[g]], rows_v.at[b], gsems[b]
      ).start()

    for b in range(NBUF):
      start_gather(b, b)

    def scale_buf(b):
      def body(i, c2):
        r0 = i * ROW_UNROLL
        for k in range(ROW_UNROLL):
          for j in range(D_MODEL // 16):
            s = pl.ds(j * 16, 16)
            rows_v[b, r0 + k, s] = rows_v[b, r0 + k, s] * SCALE
        return c2

      lax.fori_loop(0, CHUNK // ROW_UNROLL, body, 0)

    def outer(i, carry):
      for b in range(NBUF):
        g = i * NBUF + b
        off = base + g * CHUNK
        pltpu.make_async_copy(
            table_hbm.at[idx_v.at[g]], rows_v.at[b], gsems[b]
        ).wait()
        scale_buf(b)
        pltpu.make_async_copy(
            rows_v.at[b], out_hbm.at[pl.ds(off, CHUNK)], ssems[b]
        ).start()

        @pl.when(g + NBUF < NCHUNK)
        def _():
          pltpu.make_async_copy(
              rows_v.at[b], out_hbm.at[pl.ds(off, CHUNK)], ssems[b]
          ).wait()
          start_gather(b, g + NBUF)

      return carry

    lax.fori_loop(0, NCHUNK // NBUF, outer, 0)

    for t in range(NCHUNK - (NCHUNK // NBUF) * NBUF):
      g = (NCHUNK // NBUF) * NBUF + t
      off = base + g * CHUNK
      pltpu.make_async_copy(
          table_hbm.at[idx_v.at[g]], rows_v.at[t], gsems[t]
      ).wait()
      scale_buf(t)
      pltpu.make_async_copy(
          rows_v.at[t], out_hbm.at[pl.ds(off, CHUNK)], ssems[t]
      ).start()

    for b in range(NBUF):
      pltpu.make_async_copy(
          rows_v.at[b], out_hbm.at[pl.ds(base, CHUNK)], ssems[b]
      ).wait()

  return emb


_emb = _build()


@jax.jit
def kernel(x, lut):
  out = _emb(x.reshape(B_TOTAL // CHUNK, CHUNK), lut)
  return out.reshape(x.shape + (D_MODEL,))
